# Initial kernel scaffold; baseline (speedup 1.0000x reference)
import jax
import jax.numpy as jnp
from jax.experimental import pallas as pl


def kernel(x, relative_pos):
    raise NotImplementedError("write your pallas kernel here")


# speedup vs baseline: 12.2228x; 12.2228x over previous
import jax
import jax.numpy as jnp
from jax.experimental import pallas as pl

_K = 16
_BR = 256


def _knn_block(xq_ref, xk_ref, sqq_ref, skk_ref, out_ref):
    qn = xq_ref[0]
    kn = xk_ref[0]
    n = kn.shape[0]

    sqq = sqq_ref[0]
    skk = skk_ref[0]

    s = jax.lax.dot_general(
        qn, kn, (((1,), (1,)), ((), ())), preferred_element_type=jnp.float32
    )
    d2 = sqq + skk - 2.0 * s
    scores = -jnp.maximum(d2, 0.0)

    iota = jax.lax.broadcasted_iota(jnp.int32, scores.shape, 1)
    cols = []
    for _ in range(_K):
        m = jnp.max(scores, axis=1, keepdims=True)
        idx = jnp.min(jnp.where(scores == m, iota, n), axis=1)
        cols.append(idx)
        scores = jnp.where(iota == idx[:, None], -jnp.inf, scores)
    out_ref[0] = jnp.stack(cols, axis=1).astype(jnp.int32)


def kernel(x, relative_pos):
    del relative_pos
    norm = jnp.sqrt(jnp.sum(x * x, axis=1, keepdims=True))
    xn = x / jnp.maximum(norm, 1e-12)
    xt = jnp.squeeze(jnp.transpose(xn, (0, 2, 1, 3)), -1)
    b, n, d = xt.shape
    sq = jnp.sum(xt * xt, axis=-1)
    sq_q = sq[:, :, None]
    sq_k = sq[:, None, :]

    nn_idx = pl.pallas_call(
        _knn_block,
        grid=(b, n // _BR),
        in_specs=[
            pl.BlockSpec((1, _BR, d), lambda bi, i: (bi, i, 0)),
            pl.BlockSpec((1, n, d), lambda bi, i: (bi, 0, 0)),
            pl.BlockSpec((1, _BR, 1), lambda bi, i: (bi, i, 0)),
            pl.BlockSpec((1, 1, n), lambda bi, i: (bi, 0, 0)),
        ],
        out_specs=pl.BlockSpec((1, _BR, _K), lambda bi, i: (bi, i, 0)),
        out_shape=jax.ShapeDtypeStruct((b, n, _K), jnp.int32),
    )(xt, xt, sq_q, sq_k)

    center_idx = jnp.broadcast_to(
        jnp.arange(n, dtype=jnp.int32)[None, :, None], (b, n, _K)
    )
    return jnp.stack((nn_idx, center_idx), axis=0)


# speedup vs baseline: 14.0468x; 1.1492x over previous
import jax
import jax.numpy as jnp
from jax.experimental import pallas as pl

_K = 16
_BR = 256


def _knn_block(xq_ref, xk_ref, sqq_ref, skk_ref, out_ref):
    qn = xq_ref[0]
    kn = xk_ref[0]
    n = kn.shape[0]

    sqq = sqq_ref[0]
    skk = skk_ref[0]

    s = jax.lax.dot_general(
        qn, kn, (((1,), (1,)), ((), ())), preferred_element_type=jnp.float32
    )
    d2 = sqq + skk - 2.0 * s
    scores = -jnp.maximum(d2, 0.0)

    iota = jax.lax.broadcasted_iota(jnp.int32, scores.shape, 1)
    cols = []
    for _ in range(_K):
        idx = jnp.argmax(scores, axis=1).astype(jnp.int32)
        cols.append(idx)
        scores = jnp.where(iota == idx[:, None], -jnp.inf, scores)
    out_ref[0] = jnp.stack(cols, axis=1)


def kernel(x, relative_pos):
    del relative_pos
    norm = jnp.sqrt(jnp.sum(x * x, axis=1, keepdims=True))
    xn = x / jnp.maximum(norm, 1e-12)
    xt = jnp.squeeze(jnp.transpose(xn, (0, 2, 1, 3)), -1)
    b, n, d = xt.shape
    sq = jnp.sum(xt * xt, axis=-1)
    sq_q = sq[:, :, None]
    sq_k = sq[:, None, :]

    nn_idx = pl.pallas_call(
        _knn_block,
        grid=(b, n // _BR),
        in_specs=[
            pl.BlockSpec((1, _BR, d), lambda bi, i: (bi, i, 0)),
            pl.BlockSpec((1, n, d), lambda bi, i: (bi, 0, 0)),
            pl.BlockSpec((1, _BR, 1), lambda bi, i: (bi, i, 0)),
            pl.BlockSpec((1, 1, n), lambda bi, i: (bi, 0, 0)),
        ],
        out_specs=pl.BlockSpec((1, _BR, _K), lambda bi, i: (bi, i, 0)),
        out_shape=jax.ShapeDtypeStruct((b, n, _K), jnp.int32),
    )(xt, xt, sq_q, sq_k)

    center_idx = jnp.broadcast_to(
        jnp.arange(n, dtype=jnp.int32)[None, :, None], (b, n, _K)
    )
    return jnp.stack((nn_idx, center_idx), axis=0)


# speedup vs baseline: 19.8589x; 1.4138x over previous
import jax
import jax.numpy as jnp
from jax.experimental import pallas as pl

_K = 16
_BR = 256
_L = 128


def _oem_pairs(nn):
    pairs = []

    def merge(lo, m, r):
        step = r * 2
        if step < m:
            merge(lo, m, step)
            merge(lo + r, m, step)
            for i in range(lo + r, lo + m - r, step):
                pairs.append((i, i + r))
        else:
            pairs.append((lo, lo + r))

    def sort(lo, m):
        if m > 1:
            h = m // 2
            sort(lo, h)
            sort(lo + h, h)
            merge(lo, m, 1)

    sort(0, nn)
    return pairs


def _pruned_pairs(nn, need_hi):
    needed = set(range(need_hi + 1))
    kept = []
    for i, j in reversed(_oem_pairs(nn)):
        if i in needed or j in needed:
            kept.append((i, j))
            needed.add(i)
            needed.add(j)
    return list(reversed(kept))


def _knn_block(xq_ref, xk_ref, sqq_ref, skk_ref, out_ref):
    qn = xq_ref[0]
    kn = xk_ref[0]
    n = kn.shape[0]
    br = qn.shape[0]
    depths = n // _L

    sqq = sqq_ref[0]
    skk = skk_ref[0]

    s = jax.lax.dot_general(
        qn, kn, (((1,), (1,)), ((), ())), preferred_element_type=jnp.float32
    )
    d2 = sqq + skk - 2.0 * s
    scores = -jnp.maximum(d2, 0.0)

    lane = jax.lax.broadcasted_iota(jnp.int32, (br, _L), 1)
    v = [scores[:, d * _L : (d + 1) * _L] for d in range(depths)]
    dep = [jnp.full((br, _L), d, jnp.int32) for d in range(depths)]

    for i, j in _pruned_pairs(depths, _K - 1):
        sw = v[j] > v[i]
        v[i], v[j] = jnp.where(sw, v[j], v[i]), jnp.where(sw, v[i], v[j])
        dep[i], dep[j] = (
            jnp.where(sw, dep[j], dep[i]),
            jnp.where(sw, dep[i], dep[j]),
        )

    big = jnp.int32(n)
    h, hd = v[0], dep[0]
    hp = jnp.zeros((br, _L), jnp.int32)
    outs = []
    for t in range(_K):
        m = jnp.max(h, axis=1, keepdims=True)
        cand = jnp.where(h == m, hd * _L + lane, big)
        c = jnp.min(cand, axis=1, keepdims=True)
        outs.append(c)
        if t < _K - 1:
            win = cand == c
            hp = hp + win.astype(jnp.int32)
            for p in range(1, min(t + 2, depths)):
                at = win & (hp == p)
                h = jnp.where(at, v[p], h)
                hd = jnp.where(at, dep[p], hd)
    out_ref[0] = jnp.concatenate(outs, axis=1)


def kernel(x, relative_pos):
    del relative_pos
    norm = jnp.sqrt(jnp.sum(x * x, axis=1, keepdims=True))
    xn = x / jnp.maximum(norm, 1e-12)
    xt = jnp.squeeze(jnp.transpose(xn, (0, 2, 1, 3)), -1)
    b, n, d = xt.shape
    sq = jnp.sum(xt * xt, axis=-1)
    sq_q = sq[:, :, None]
    sq_k = sq[:, None, :]

    nn_idx = pl.pallas_call(
        _knn_block,
        grid=(b, n // _BR),
        in_specs=[
            pl.BlockSpec((1, _BR, d), lambda bi, i: (bi, i, 0)),
            pl.BlockSpec((1, n, d), lambda bi, i: (bi, 0, 0)),
            pl.BlockSpec((1, _BR, 1), lambda bi, i: (bi, i, 0)),
            pl.BlockSpec((1, 1, n), lambda bi, i: (bi, 0, 0)),
        ],
        out_specs=pl.BlockSpec((1, _BR, _K), lambda bi, i: (bi, i, 0)),
        out_shape=jax.ShapeDtypeStruct((b, n, _K), jnp.int32),
    )(xt, xt, sq_q, sq_k)

    center_idx = jnp.broadcast_to(
        jnp.arange(n, dtype=jnp.int32)[None, :, None], (b, n, _K)
    )
    return jnp.stack((nn_idx, center_idx), axis=0)


# speedup vs baseline: 20.0046x; 1.0073x over previous
import jax
import jax.numpy as jnp
from jax.experimental import pallas as pl

_K = 16
_BR = 256
_L = 128


def _oem_pairs(nn):
    pairs = []

    def merge(lo, m, r):
        step = r * 2
        if step < m:
            merge(lo, m, step)
            merge(lo + r, m, step)
            for i in range(lo + r, lo + m - r, step):
                pairs.append((i, i + r))
        else:
            pairs.append((lo, lo + r))

    def sort(lo, m):
        if m > 1:
            h = m // 2
            sort(lo, h)
            sort(lo + h, h)
            merge(lo, m, 1)

    sort(0, nn)
    return pairs


def _pruned_pairs(nn, need_hi):
    needed = set(range(need_hi + 1))
    kept = []
    for i, j in reversed(_oem_pairs(nn)):
        if i in needed or j in needed:
            kept.append((i, j))
            needed.add(i)
            needed.add(j)
    return list(reversed(kept))


def _select(scores, n):
    br = scores.shape[0]
    depths = n // _L
    lane = jax.lax.broadcasted_iota(jnp.int32, (br, _L), 1)
    v = [scores[:, d * _L : (d + 1) * _L] for d in range(depths)]
    dep = [jnp.full((br, _L), d, jnp.int32) for d in range(depths)]

    for i, j in _pruned_pairs(depths, _K - 1):
        sw = v[j] < v[i]
        v[i], v[j] = jnp.where(sw, v[j], v[i]), jnp.where(sw, v[i], v[j])
        dep[i], dep[j] = (
            jnp.where(sw, dep[j], dep[i]),
            jnp.where(sw, dep[i], dep[j]),
        )

    big = jnp.int32(n)
    h, hd = v[0], dep[0]
    hp = jnp.zeros((br, _L), jnp.int32)
    outs = []
    for t in range(_K):
        m = jnp.min(h, axis=1, keepdims=True)
        cand = jnp.where(h == m, hd * _L + lane, big)
        c = jnp.min(cand, axis=1, keepdims=True)
        outs.append(c)
        if t < _K - 1:
            win = cand == c
            hp = hp + win.astype(jnp.int32)
            for p in range(1, min(t + 2, depths)):
                at = win & (hp == p)
                h = jnp.where(at, v[p], h)
                hd = jnp.where(at, dep[p], hd)
    return jnp.concatenate(outs, axis=1)


_G = 64


def _knn_block(xq_ref, xk_ref, sqq_ref, skk_ref, out_ref):
    qn = xq_ref[0]
    kn = xk_ref[0]
    n = kn.shape[0]
    br = qn.shape[0]

    sqq = sqq_ref[0]
    skk = skk_ref[0]

    s2 = jax.lax.dot_general(
        qn, kn, (((1,), (1,)), ((), ())), preferred_element_type=jnp.float32
    )
    d2 = sqq + skk - s2
    scores = jnp.maximum(d2, 0.0)

    outs = [
        _select(scores[g * _G : (g + 1) * _G], n) for g in range(br // _G)
    ]
    out_ref[0] = jnp.concatenate(outs, axis=0)


def kernel(x, relative_pos):
    del relative_pos
    norm = jnp.sqrt(jnp.sum(x * x, axis=1, keepdims=True))
    xn = x / jnp.maximum(norm, 1e-12)
    xt = jnp.squeeze(jnp.transpose(xn, (0, 2, 1, 3)), -1)
    b, n, d = xt.shape
    sq = jnp.sum(xt * xt, axis=-1)
    sq_q = sq[:, :, None]
    sq_k = sq[:, None, :]

    nn_idx = pl.pallas_call(
        _knn_block,
        grid=(b, n // _BR),
        in_specs=[
            pl.BlockSpec((1, _BR, d), lambda bi, i: (bi, i, 0)),
            pl.BlockSpec((1, n, d), lambda bi, i: (bi, 0, 0)),
            pl.BlockSpec((1, _BR, 1), lambda bi, i: (bi, i, 0)),
            pl.BlockSpec((1, 1, n), lambda bi, i: (bi, 0, 0)),
        ],
        out_specs=pl.BlockSpec((1, _BR, _K), lambda bi, i: (bi, i, 0)),
        out_shape=jax.ShapeDtypeStruct((b, n, _K), jnp.int32),
    )(2.0 * xt, xt, sq_q, sq_k)

    center_idx = jnp.broadcast_to(
        jnp.arange(n, dtype=jnp.int32)[None, :, None], (b, n, _K)
    )
    return jnp.stack((nn_idx, center_idx), axis=0)
